# baseline (speedup 1.0000x reference)
import functools

import jax
import jax.numpy as jnp
from jax import lax
from jax.experimental import pallas as pl
from jax.experimental.pallas import tpu as pltpu
from jax.experimental.pallas import tpu_sc as plsc

_M = 0.25
_GAMMA = 64.0
_EPS = 1e-6
_D = 131072
_NS = 16
_CHUNK = _D // _NS
_UNROLL = 8


def _sc_loss_terms(anchor4d, negative4d, labels2):
    mesh = plsc.VectorSubcoreMesh(core_axis_name="c", subcore_axis_name="s")

    @functools.partial(
        pl.kernel,
        out_type=jax.ShapeDtypeStruct((2, 16), jnp.float32),
        mesh=mesh,
        scratch_types=[
            pltpu.VMEM((16,), jnp.int32),
            pltpu.VMEM((_CHUNK,), jnp.float32),
            pltpu.VMEM((_CHUNK,), jnp.float32),
            pltpu.VMEM((16,), jnp.float32),
            pltpu.VMEM((16,), jnp.int32),
            pltpu.VMEM_SHARED((16,), jnp.float32),
        ] + [pltpu.SemaphoreType.DMA] * 8,
    )
    def body(a_hbm, n_hbm, lab_hbm, out_hbm, lab_v, a_v, n_v,
             acc_v, eidx_v, shared, *sems):
        cid = lax.axis_index("c")
        sid = lax.axis_index("s")
        pltpu.sync_copy(lab_hbm, lab_v.at[pl.ds(0, 2)])
        @pl.when(sid == 0)
        def _():
            acc_v[...] = jnp.zeros((16,), jnp.float32)
            pltpu.sync_copy(acc_v, shared)
        lv = lab_v[...]
        e = (lv[0] != lv[1]).astype(jnp.int32)
        i = cid * e
        j = (1 - cid) * e
        off = sid * _CHUNK
        q = _CHUNK // 4
        cps = []
        for p in range(4):
            sa, sn = sems[2 * p], sems[2 * p + 1]
            cps.append((
                pltpu.async_copy(
                    a_hbm.at[i, j, j, pl.ds(off + p * q, q)],
                    a_v.at[pl.ds(p * q, q)], sa),
                pltpu.async_copy(
                    n_hbm.at[i, j, j, pl.ds(off + p * q, q)],
                    n_v.at[pl.ds(p * q, q)], sn),
            ))

        def step(i, accs):
            base = pl.multiple_of(i * (16 * _UNROLL), 16 * _UNROLL)
            new = []
            for j in range(_UNROLL):
                a = a_v[pl.ds(base + j * 16, 16)]
                n = n_v[pl.ds(base + j * 16, 16)]
                d = a - n + _EPS
                new.append(accs[j] + d * d)
            return tuple(new)

        accs = (jnp.zeros((16,), jnp.float32),) * _UNROLL
        nsteps = q // (16 * _UNROLL)
        for p in range(4):
            cp_a, cp_n = cps[p]
            cp_a.wait()
            cp_n.wait()
            accs = lax.fori_loop(p * nsteps, (p + 1) * nsteps, step, accs)
        acc = accs[0]
        for j in range(1, _UNROLL):
            acc = acc + accs[j]
        acc_v[...] = acc
        eidx_v[...] = lax.iota(jnp.int32, 16)
        plsc.subcore_barrier()
        pltpu.sync_copy(acc_v, shared.at[eidx_v], add=True)
        plsc.subcore_barrier()

        @pl.when(sid == 0)
        def _():
            pltpu.sync_copy(shared, acc_v)
            tot = acc_v[...]
            lane = lax.iota(jnp.int32, 16)
            for sh in (8, 4, 2, 1):
                tot = tot + tot.at[lane ^ sh].get(mode="promise_in_bounds")
            y16 = jnp.maximum(_GAMMA * (tot - _M * _M), 0.0)
            t = jnp.exp(-y16)
            z = t * (1.0 - t * (0.5 - t * (1.0 / 3.0 - t * 0.25)))
            for _ in range(3):
                z = z - 1.0 + (1.0 + t) * jnp.exp(-z)
            acc_v[...] = y16 + z
            pltpu.sync_copy(acc_v, out_hbm.at[cid])

    return body(anchor4d, negative4d, labels2)


def kernel(anchor, positive, negative, labels):
    del positive
    terms = _sc_loss_terms(anchor, negative, labels.astype(jnp.int32))
    return 0.5 * (terms[0, 0] + terms[1, 0])


# speedup vs baseline: 1.0049x; 1.0049x over previous
import functools

import jax
import jax.numpy as jnp
from jax import lax
from jax.experimental import pallas as pl
from jax.experimental.pallas import tpu as pltpu
from jax.experimental.pallas import tpu_sc as plsc

_M = 0.25
_GAMMA = 64.0
_EPS = 1e-6
_D = 131072
_NS = 16
_CHUNK = _D // _NS
_UNROLL = 8


def _sc_loss_terms(anchor4d, negative4d, labels2):
    mesh = plsc.VectorSubcoreMesh(core_axis_name="c", subcore_axis_name="s")

    @functools.partial(
        pl.kernel,
        out_type=jax.ShapeDtypeStruct((2, 16), jnp.float32),
        mesh=mesh,
        scratch_types=[
            pltpu.VMEM((16,), jnp.int32),
            pltpu.VMEM((_CHUNK,), jnp.float32),
            pltpu.VMEM((_CHUNK,), jnp.float32),
            pltpu.VMEM((16,), jnp.float32),
            pltpu.VMEM((16,), jnp.int32),
            pltpu.VMEM_SHARED((16,), jnp.float32),
        ] + [pltpu.SemaphoreType.DMA] * 4,
    )
    def body(a_hbm, n_hbm, lab_hbm, out_hbm, lab_v, a_v, n_v,
             acc_v, eidx_v, shared, *sems):
        cid = lax.axis_index("c")
        sid = lax.axis_index("s")
        pltpu.sync_copy(lab_hbm, lab_v.at[pl.ds(0, 2)])
        @pl.when(sid == 0)
        def _():
            acc_v[...] = jnp.zeros((16,), jnp.float32)
            pltpu.sync_copy(acc_v, shared)
        lv = lab_v[...]
        e = (lv[0] != lv[1]).astype(jnp.int32)
        i = cid * e
        j = (1 - cid) * e
        off = sid * _CHUNK
        half = _CHUNK // 2
        cp_a0 = pltpu.async_copy(
            a_hbm.at[i, j, j, pl.ds(off, half)],
            a_v.at[pl.ds(0, half)], sems[0])
        cp_n0 = pltpu.async_copy(
            n_hbm.at[i, j, j, pl.ds(off, half)],
            n_v.at[pl.ds(0, half)], sems[1])
        cp_a1 = pltpu.async_copy(
            a_hbm.at[i, j, j, pl.ds(off + half, half)],
            a_v.at[pl.ds(half, half)], sems[2])
        cp_n1 = pltpu.async_copy(
            n_hbm.at[i, j, j, pl.ds(off + half, half)],
            n_v.at[pl.ds(half, half)], sems[3])

        def step(i, accs):
            base = pl.multiple_of(i * (16 * _UNROLL), 16 * _UNROLL)
            new = []
            for j in range(_UNROLL):
                a = a_v[pl.ds(base + j * 16, 16)]
                n = n_v[pl.ds(base + j * 16, 16)]
                d = a - n + _EPS
                new.append(accs[j] + d * d)
            return tuple(new)

        nsteps = half // (16 * _UNROLL)
        cp_a0.wait()
        cp_n0.wait()
        accs = lax.fori_loop(0, nsteps, step,
                             (jnp.zeros((16,), jnp.float32),) * _UNROLL)
        cp_a1.wait()
        cp_n1.wait()
        accs = lax.fori_loop(nsteps, 2 * nsteps, step, accs)
        acc = accs[0]
        for j in range(1, _UNROLL):
            acc = acc + accs[j]
        acc_v[...] = acc
        eidx_v[...] = lax.iota(jnp.int32, 16)
        plsc.subcore_barrier()
        pltpu.sync_copy(acc_v, shared.at[eidx_v], add=True)
        plsc.subcore_barrier()

        @pl.when(sid == 0)
        def _():
            pltpu.sync_copy(shared, acc_v)
            tot = acc_v[...]
            lane = lax.iota(jnp.int32, 16)
            for sh in (8, 4, 2, 1):
                tot = tot + tot.at[lane ^ sh].get(mode="promise_in_bounds")
            y16 = jnp.maximum(_GAMMA * (tot - _M * _M), 0.0)
            t = jnp.exp(-y16)
            z = t * (1.0 - t * (0.5 - t * (1.0 / 3.0 - t * 0.25)))
            for _ in range(3):
                z = z - 1.0 + (1.0 + t) * jnp.exp(-z)
            acc_v[...] = y16 + z
            pltpu.sync_copy(acc_v, out_hbm.at[cid])

    return body(anchor4d, negative4d, labels2)


def kernel(anchor, positive, negative, labels):
    del positive
    terms = _sc_loss_terms(anchor, negative, labels.astype(jnp.int32))
    return 0.5 * (terms[0, 0] + terms[1, 0])
